# baseline (speedup 1.0000x reference)
import dataclasses
import functools
import math

import jax
import jax.numpy as jnp
from jax import lax
from jax.experimental import pallas as pl
from jax.experimental.pallas import tpu as pltpu
from jax.experimental.pallas import tpu_sc as plsc

B = 8
H = 512
W = 512
C = 3
K = 256

TWO_PI = 2.0 * math.pi

TOTAL_WORDS = B * H * W * C
LANES = 128
FLAT_ROWS = TOTAL_WORDS // LANES
ROWS_PER_BATCH = FLAT_ROWS // B
TC_BLOCK_ROWS = 768
TC_BLOCKS_PER_BATCH = ROWS_PER_BATCH // TC_BLOCK_ROWS

SC_LANES = 16
SC_BLOCK = 8192
SC_UNROLL = 8


def _up(v):
    return jnp.concatenate([v[1:, :], v[:1, :]], axis=0)


def _down(v):
    return jnp.concatenate([v[-1:, :], v[:-1, :]], axis=0)


def _shl(v, k):
    return jnp.concatenate([v[:, k:], _up(v)[:, :k]], axis=1)


def _shr(v, k):
    return jnp.concatenate([_down(v)[:, LANES - k:], v[:, :LANES - k]], axis=1)


def _tc_body(x_ref, par_ref, out_ref):
    x = x_ref[...]
    nqx = par_ref[0, 0, 0]
    nqy = par_ref[0, 0, 1]
    nqz = par_ref[0, 0, 2]
    qq = par_ref[0, 0, 3]
    scale = par_ref[0, 0, 4]

    shape = (TC_BLOCK_ROWS, LANES)
    flat_idx = (lax.broadcasted_iota(jnp.int32, shape, 0) * 2
                + lax.broadcasted_iota(jnp.int32, shape, 1))
    phase = flat_idx % 3
    m0 = phase == 0
    m1 = phase == 1

    qv = jnp.where(m0, nqx, jnp.where(m1, nqy, nqz))
    sq = x * x
    cv = x * qv
    s3 = sq + _shl(sq, 1) + _shl(sq, 2)
    a3 = cv + _shl(cv, 1) + _shl(cv, 2)
    s = jnp.where(m0, s3, jnp.where(m1, _shr(s3, 1), _shr(s3, 2)))
    a = jnp.where(m0, a3, jnp.where(m1, _shr(a3, 1), _shr(a3, 2)))

    mag = s * qq
    a2 = a * a
    bcd_sq = jnp.maximum(mag - a2, 0.0)
    real = a2 - bcd_sq
    imag = jnp.sqrt(bcd_sq) * a * 2.0
    u = jnp.arctan2(imag, real) / TWO_PI + 0.5
    idx = jnp.floor(u * scale).astype(jnp.int32) & (K - 1)
    out_ref[...] = (lax.bitcast_convert_type(mag, jnp.int32) & (-1024)) | (
        idx * 3 + phase
    )


def _tc_stage(flat_words, params):
    return pl.pallas_call(
        _tc_body,
        grid=(B * TC_BLOCKS_PER_BATCH,),
        in_specs=[
            pl.BlockSpec((TC_BLOCK_ROWS, LANES), lambda g: (g, 0)),
            pl.BlockSpec((1, 1, 8), lambda g: (g // TC_BLOCKS_PER_BATCH, 0, 0),
                         memory_space=pltpu.SMEM),
        ],
        out_specs=pl.BlockSpec((TC_BLOCK_ROWS, LANES), lambda g: (g, 0)),
        out_shape=jax.ShapeDtypeStruct((FLAT_ROWS, LANES), jnp.int32),
    )(flat_words, params)


def _sc_stage(packed1d, tab):
    mesh = plsc.VectorSubcoreMesh(core_axis_name="c", subcore_axis_name="s")
    cp = pltpu.CompilerParams()
    if "needs_layout_passes" in pltpu.CompilerParams.__dataclass_fields__:
        cp = dataclasses.replace(cp, needs_layout_passes=False)

    @functools.partial(
        pl.kernel,
        out_type=jax.ShapeDtypeStruct((TOTAL_WORDS,), jnp.float32),
        mesh=mesh,
        scratch_types=[pltpu.VMEM((C * K,), jnp.float32)],
        compiler_params=cp,
    )
    def sc_kernel(in_hbm, tab_hbm, out_hbm, tab_v):
        pltpu.sync_copy(tab_hbm, tab_v)

        def body(in_v, out_v):
            def chunk(base):
                for u in range(SC_UNROLL):
                    slc = pl.ds(base + u * SC_LANES, SC_LANES)
                    w = in_v[slc]
                    kidx = w & 1023
                    m = plsc.bitcast(w & (-1024), jnp.float32)
                    g = plsc.load_gather(tab_v, [kidx])
                    out_v[slc] = g * m

            pl.loop(0, SC_BLOCK, step=SC_LANES * SC_UNROLL)(chunk)

        pltpu.emit_pipeline(
            body,
            grid=(TOTAL_WORDS // SC_BLOCK,),
            in_specs=[pl.BlockSpec((SC_BLOCK,), index_map=lambda i: (i,))],
            out_specs=[pl.BlockSpec((SC_BLOCK,), index_map=lambda i: (i,))],
            core_axis_name=("c", "s"),
            dimension_semantics=(pltpu.PARALLEL,),
        )(in_hbm, out_hbm)

    return sc_kernel(packed1d, tab)


def kernel(camera_orientation_conj, surface_normals, cyclic_colourmap, degree):
    q = camera_orientation_conj.reshape(B, 4)
    nq = -q[:, 1:4]
    qq = jnp.sum(q * q, axis=1, keepdims=True)
    scale = jnp.full((B, 1), degree * K, dtype=jnp.float32)
    pad = jnp.zeros((B, 3), dtype=jnp.float32)
    params = jnp.concatenate([nq, qq, scale, pad], axis=1).reshape(B, 1, 8)

    flat_words = surface_normals.reshape(FLAT_ROWS, LANES)
    packed = _tc_stage(flat_words, params)
    packed1d = packed.reshape(TOTAL_WORDS)
    tab = cyclic_colourmap.reshape(C * K)

    out1d = _sc_stage(packed1d, tab)
    return out1d.reshape(B, H, W, C)


# speedup vs baseline: 28.9351x; 28.9351x over previous
import dataclasses
import functools
import math

import jax
import jax.numpy as jnp
from jax import lax
from jax.experimental import pallas as pl
from jax.experimental.pallas import tpu as pltpu
from jax.experimental.pallas import tpu_sc as plsc

B = 8
H = 512
W = 512
C = 3
K = 256

TWO_PI = 2.0 * math.pi

PLANES = B * C
PLANE_ROWS = H
TC_R = 128
TC_RB = PLANE_ROWS // TC_R

SC_R = 8
SC_CHUNKS = PLANE_ROWS // SC_R
SC_LANES = 16
SC_UNROLL = 4


def _tc_body(x_ref, y_ref, z_ref, par_ref, out_ref):
    x = x_ref[...]
    y = y_ref[...]
    z = z_ref[...]
    nqx = par_ref[0, 0, 0]
    nqy = par_ref[0, 0, 1]
    nqz = par_ref[0, 0, 2]
    qq = par_ref[0, 0, 3]
    scale = par_ref[0, 0, 4]

    s = x * x + y * y + z * z
    a = x * nqx + y * nqy + z * nqz
    mag = s * qq
    a2 = a * a
    bcd_sq = jnp.maximum(mag - a2, 0.0)
    real = a2 - bcd_sq
    imag = jnp.sqrt(bcd_sq) * a * 2.0
    u = jnp.arctan2(imag, real) / TWO_PI + 0.5
    idx = jnp.floor(u * scale).astype(jnp.int32) & (K - 1)
    out_ref[...] = (lax.bitcast_convert_type(mag, jnp.int32) & (-256)) | idx


def _tc_stage(planes, params):
    def plane_map(c):
        return lambda b, r: ((3 * b + c) * TC_RB + r, 0)

    return pl.pallas_call(
        _tc_body,
        grid=(B, TC_RB),
        in_specs=[
            pl.BlockSpec((TC_R, W), plane_map(0)),
            pl.BlockSpec((TC_R, W), plane_map(1)),
            pl.BlockSpec((TC_R, W), plane_map(2)),
            pl.BlockSpec((1, 1, 8), lambda b, r: (b, 0, 0),
                         memory_space=pltpu.SMEM),
        ],
        out_specs=pl.BlockSpec((TC_R, W), lambda b, r: (b * TC_RB + r, 0)),
        out_shape=jax.ShapeDtypeStruct((B * PLANE_ROWS, W), jnp.int32),
    )(planes, planes, planes, params)


def _sc_stage(packed, tab):
    mesh = plsc.VectorSubcoreMesh(core_axis_name="c", subcore_axis_name="s")
    cp = pltpu.CompilerParams()
    if "needs_layout_passes" in pltpu.CompilerParams.__dataclass_fields__:
        cp = dataclasses.replace(cp, needs_layout_passes=False)

    def out_map(c):
        return lambda j: ((3 * (j // SC_CHUNKS) + c) * SC_CHUNKS
                          + (j % SC_CHUNKS), 0)

    @functools.partial(
        pl.kernel,
        out_type=jax.ShapeDtypeStruct((PLANES * PLANE_ROWS, W), jnp.float32),
        mesh=mesh,
        scratch_types=[pltpu.VMEM((C * K,), jnp.float32)],
        compiler_params=cp,
    )
    def sc_kernel(in_hbm, tab_hbm, out_hbm, tab_v):
        pltpu.sync_copy(tab_hbm, tab_v)

        def body(in_v, ox_v, oy_v, oz_v):
            for r in range(SC_R):
                def chunk(base, r=r):
                    for u in range(SC_UNROLL):
                        slc = pl.ds(base + u * SC_LANES, SC_LANES)
                        w = in_v[r, slc]
                        kidx = w & (K - 1)
                        m = plsc.bitcast(w & (-256), jnp.float32)
                        ox_v[r, slc] = plsc.load_gather(tab_v, [kidx]) * m
                        oy_v[r, slc] = plsc.load_gather(tab_v, [kidx + K]) * m
                        oz_v[r, slc] = plsc.load_gather(tab_v, [kidx + 2 * K]) * m

                pl.loop(0, W, step=SC_LANES * SC_UNROLL)(chunk)

        pltpu.emit_pipeline(
            body,
            grid=(B * SC_CHUNKS,),
            in_specs=[pl.BlockSpec((SC_R, W), index_map=lambda j: (j, 0))],
            out_specs=[
                pl.BlockSpec((SC_R, W), index_map=out_map(0)),
                pl.BlockSpec((SC_R, W), index_map=out_map(1)),
                pl.BlockSpec((SC_R, W), index_map=out_map(2)),
            ],
            core_axis_name=("c", "s"),
            dimension_semantics=(pltpu.PARALLEL,),
        )(in_hbm, out_hbm, out_hbm, out_hbm)

    return sc_kernel(packed, tab)


def kernel(camera_orientation_conj, surface_normals, cyclic_colourmap, degree):
    q = camera_orientation_conj.reshape(B, 4)
    nq = -q[:, 1:4]
    qq = jnp.sum(q * q, axis=1, keepdims=True)
    scale = jnp.full((B, 1), degree * K, dtype=jnp.float32)
    pad = jnp.zeros((B, 3), dtype=jnp.float32)
    params = jnp.concatenate([nq, qq, scale, pad], axis=1).reshape(B, 1, 8)

    planes = surface_normals.transpose(0, 3, 1, 2).reshape(PLANES * PLANE_ROWS, W)
    packed = _tc_stage(planes, params)

    tab = cyclic_colourmap.transpose(1, 0).reshape(C * K)
    out2d = _sc_stage(packed, tab)
    return out2d.reshape(B, C, H, W).transpose(0, 2, 3, 1)


# speedup vs baseline: 29.0895x; 1.0053x over previous
import dataclasses
import functools
import math

import jax
import jax.numpy as jnp
from jax import lax
from jax.experimental import pallas as pl
from jax.experimental.pallas import tpu as pltpu
from jax.experimental.pallas import tpu_sc as plsc

B = 8
H = 512
W = 512
C = 3
K = 256

TWO_PI = 2.0 * math.pi

PLANES = B * C
PLANE_ROWS = H
TC_R = 128
TC_RB = PLANE_ROWS // TC_R

SC_R = 8
SC_CHUNKS = PLANE_ROWS // SC_R
SC_LANES = 16
SC_UNROLL = 8


def _tc_body(x_ref, y_ref, z_ref, par_ref, out_ref):
    x = x_ref[...]
    y = y_ref[...]
    z = z_ref[...]
    nqx = par_ref[0, 0, 0]
    nqy = par_ref[0, 0, 1]
    nqz = par_ref[0, 0, 2]
    qq = par_ref[0, 0, 3]
    scale = par_ref[0, 0, 4]

    s = x * x + y * y + z * z
    a = x * nqx + y * nqy + z * nqz
    mag = s * qq
    a2 = a * a
    bcd_sq = jnp.maximum(mag - a2, 0.0)
    real = a2 - bcd_sq
    imag = jnp.sqrt(bcd_sq) * a * 2.0
    u = jnp.arctan2(imag, real) / TWO_PI + 0.5
    idx = jnp.floor(u * scale).astype(jnp.int32) & (K - 1)
    out_ref[...] = (lax.bitcast_convert_type(mag, jnp.int32) & (-256)) | idx


def _tc_stage(planes, params):
    def plane_map(c):
        return lambda b, r: ((3 * b + c) * TC_RB + r, 0)

    return pl.pallas_call(
        _tc_body,
        grid=(B, TC_RB),
        in_specs=[
            pl.BlockSpec((TC_R, W), plane_map(0)),
            pl.BlockSpec((TC_R, W), plane_map(1)),
            pl.BlockSpec((TC_R, W), plane_map(2)),
            pl.BlockSpec((1, 1, 8), lambda b, r: (b, 0, 0),
                         memory_space=pltpu.SMEM),
        ],
        out_specs=pl.BlockSpec((TC_R, W), lambda b, r: (b * TC_RB + r, 0)),
        out_shape=jax.ShapeDtypeStruct((B * PLANE_ROWS, W), jnp.int32),
    )(planes, planes, planes, params)


def _sc_stage(packed, tabx, taby, tabz):
    mesh = plsc.VectorSubcoreMesh(core_axis_name="c", subcore_axis_name="s")
    cp = pltpu.CompilerParams()
    if "needs_layout_passes" in pltpu.CompilerParams.__dataclass_fields__:
        cp = dataclasses.replace(cp, needs_layout_passes=False)

    def out_map(c):
        return lambda j: ((3 * (j // SC_CHUNKS) + c) * SC_CHUNKS
                          + (j % SC_CHUNKS), 0)

    @functools.partial(
        pl.kernel,
        out_type=jax.ShapeDtypeStruct((PLANES * PLANE_ROWS, W), jnp.float32),
        mesh=mesh,
        scratch_types=[pltpu.VMEM((K,), jnp.float32)] * 3,
        compiler_params=cp,
    )
    def sc_kernel(in_hbm, tx_hbm, ty_hbm, tz_hbm, out_hbm, tx_v, ty_v, tz_v):
        pltpu.sync_copy(tx_hbm, tx_v)
        pltpu.sync_copy(ty_hbm, ty_v)
        pltpu.sync_copy(tz_hbm, tz_v)

        def body(in_v, ox_v, oy_v, oz_v):
            for r in range(SC_R):
                def chunk(base, r=r):
                    for u in range(SC_UNROLL):
                        slc = pl.ds(base + u * SC_LANES, SC_LANES)
                        w = in_v[r, slc]
                        kidx = w & (K - 1)
                        m = plsc.bitcast(w & (-256), jnp.float32)
                        ox_v[r, slc] = plsc.load_gather(tx_v, [kidx]) * m
                        oy_v[r, slc] = plsc.load_gather(ty_v, [kidx]) * m
                        oz_v[r, slc] = plsc.load_gather(tz_v, [kidx]) * m

                pl.loop(0, W, step=SC_LANES * SC_UNROLL)(chunk)

        pltpu.emit_pipeline(
            body,
            grid=(B * SC_CHUNKS,),
            in_specs=[pl.BlockSpec((SC_R, W), index_map=lambda j: (j, 0))],
            out_specs=[
                pl.BlockSpec((SC_R, W), index_map=out_map(0)),
                pl.BlockSpec((SC_R, W), index_map=out_map(1)),
                pl.BlockSpec((SC_R, W), index_map=out_map(2)),
            ],
            core_axis_name=("c", "s"),
            dimension_semantics=(pltpu.PARALLEL,),
        )(in_hbm, out_hbm, out_hbm, out_hbm)

    return sc_kernel(packed, tabx, taby, tabz)


def kernel(camera_orientation_conj, surface_normals, cyclic_colourmap, degree):
    q = camera_orientation_conj.reshape(B, 4)
    nq = -q[:, 1:4]
    qq = jnp.sum(q * q, axis=1, keepdims=True)
    scale = jnp.full((B, 1), degree * K, dtype=jnp.float32)
    pad = jnp.zeros((B, 3), dtype=jnp.float32)
    params = jnp.concatenate([nq, qq, scale, pad], axis=1).reshape(B, 1, 8)

    planes = surface_normals.transpose(0, 3, 1, 2).reshape(PLANES * PLANE_ROWS, W)
    packed = _tc_stage(planes, params)

    out2d = _sc_stage(packed, cyclic_colourmap[:, 0], cyclic_colourmap[:, 1],
                      cyclic_colourmap[:, 2])
    return out2d.reshape(B, C, H, W).transpose(0, 2, 3, 1)
